# baseline (speedup 1.0000x reference)
import functools

import jax
import jax.numpy as jnp
from jax import lax
from jax.experimental import pallas as pl
from jax.experimental.pallas import tpu as pltpu
from jax.experimental.pallas import tpu_sc as plsc

_NC = 2
_NS = 16
_L = 16
_NW = _NC * _NS

_B = 4
_S = 2048
_D = 1024
_ROWS = _B * _S
_SEG = _ROWS // _NW
_WPR = _S // _SEG
_CHUNK = 16
_NCHUNK = _SEG // _CHUNK
_NBUF = 3


def _body(x_hbm, ids_hbm, table_hbm, out_hbm,
          ids_v, pid_v, xb0, xb1, xb2, rb0, rb1, rb2,
          sx0, sx1, sx2, sg0, sg1, sg2, ss0, ss1, ss2, si):
    xbufs = (xb0, xb1, xb2)
    rbufs = (rb0, rb1, rb2)
    sx = (sx0, sx1, sx2)
    sg = (sg0, sg1, sg2)
    ss = (ss0, ss1, ss2)

    wid = lax.axis_index("s") * _NC + lax.axis_index("c")
    b = wid // _WPR
    s = wid % _WPR

    seg_base = wid * _SEG
    p0 = s * _SEG

    def x_load(ch):
        sl = ch % _NBUF
        return pltpu.async_copy(
            x_hbm.at[pl.ds(seg_base + ch * _CHUNK, _CHUNK)], xbufs[sl], sx[sl])

    def g_load(ch):
        sl = ch % _NBUF
        return pltpu.async_copy(
            table_hbm.at[pid_v.at[pl.ds(p0 + ch * _CHUNK, _CHUNK)]],
            rbufs[sl], sg[sl])

    d_ids = pltpu.async_copy(ids_hbm.at[b], ids_v, si)
    dx01 = [x_load(0), x_load(1)]
    d_ids.wait()

    def cnt_step(j, carry):
        v = ids_v[pl.ds(j * _L, _L)]
        return carry + plsc.all_reduce_population_count(v != 0)

    cnt = lax.fori_loop(0, s * (_SEG // _L), cnt_step,
                        jnp.zeros((_L,), jnp.int32))

    def pid_step(j, carry):
        v = ids_v[pl.ds(j * _L, _L)]
        m = jnp.where(v != 0, jnp.int32(1), jnp.int32(0))
        c = plsc.cumsum(m) + carry
        pid_v[pl.ds(j * _L, _L)] = c * m
        return jnp.max(c)

    lax.fori_loop(s * (_SEG // _L), (s + 1) * (_SEG // _L), pid_step,
                  jnp.max(cnt))

    def accumulate(sl):
        xb, rb = xbufs[sl], rbufs[sl]

        def row_body(r, _):
            @plsc.parallel_loop(0, _D // _L, unroll=8)
            def _k(k):
                plsc.addupdate(xb.at[r, pl.ds(k * _L, _L)],
                               rb[r, k // 8, pl.ds((k % 8) * _L, _L)])
            return 0

        lax.fori_loop(0, _CHUNK, row_body, 0)

    l_pend = [None] * _NBUF
    s_pend = [None] * _NBUF
    for ch in range(2):
        l_pend[ch % _NBUF] = (dx01[ch], g_load(ch))
    for ch in range(_NCHUNK):
        sl = ch % _NBUF
        dx, dg = l_pend[sl]
        dx.wait()
        dg.wait()
        l_pend[sl] = None
        if ch + 2 < _NCHUNK:
            s2 = (ch + 2) % _NBUF
            if s_pend[s2] is not None:
                s_pend[s2].wait()
                s_pend[s2] = None
            l_pend[s2] = (x_load(ch + 2), g_load(ch + 2))
        accumulate(sl)
        s_pend[sl] = pltpu.async_copy(
            xbufs[sl], out_hbm.at[pl.ds(seg_base + ch * _CHUNK, _CHUNK)],
            ss[sl])
    for d in s_pend:
        if d is not None:
            d.wait()


_pe = functools.partial(
    pl.kernel,
    out_type=jax.ShapeDtypeStruct((_ROWS, _D), jnp.float32),
    mesh=plsc.VectorSubcoreMesh(
        core_axis_name="c", subcore_axis_name="s",
        num_cores=_NC, num_subcores=_NS),
    compiler_params=pltpu.CompilerParams(needs_layout_passes=False),
    scratch_types=[
        pltpu.VMEM((_S,), jnp.int32),
        pltpu.VMEM((_S,), jnp.int32),
        pltpu.VMEM((_CHUNK, _D), jnp.float32),
        pltpu.VMEM((_CHUNK, _D), jnp.float32),
        pltpu.VMEM((_CHUNK, _D), jnp.float32),
        pltpu.VMEM((_CHUNK, _D // 128, 128), jnp.float32),
        pltpu.VMEM((_CHUNK, _D // 128, 128), jnp.float32),
        pltpu.VMEM((_CHUNK, _D // 128, 128), jnp.float32),
        pltpu.SemaphoreType.DMA,
        pltpu.SemaphoreType.DMA,
        pltpu.SemaphoreType.DMA,
        pltpu.SemaphoreType.DMA,
        pltpu.SemaphoreType.DMA,
        pltpu.SemaphoreType.DMA,
        pltpu.SemaphoreType.DMA,
        pltpu.SemaphoreType.DMA,
        pltpu.SemaphoreType.DMA,
        pltpu.SemaphoreType.DMA,
    ],
)(_body)


def kernel(X, input_ids, table):
    out = _pe(X.reshape(_ROWS, _D), input_ids,
              table.reshape(table.shape[0], _D // 128, 128))
    return out.reshape(_B, _S, _D)


# speedup vs baseline: 1.3065x; 1.3065x over previous
import functools

import jax
import jax.numpy as jnp
from jax import lax
from jax.experimental import pallas as pl
from jax.experimental.pallas import tpu as pltpu
from jax.experimental.pallas import tpu_sc as plsc

_NC = 2
_NS = 16
_L = 16
_NW = _NC * _NS

_B = 4
_S = 2048
_D = 1024
_ROWS = _B * _S
_SEG = _ROWS // _NW
_WPR = _S // _SEG
_CHUNK = 16
_NCHUNK = _SEG // _CHUNK
_NBUF = 3


def _body(x_hbm, ids_hbm, table_hbm, out_hbm,
          ids_v, pid_v, xb0, xb1, xb2, rb0, rb1, rb2,
          sx0, sx1, sx2, sg0, sg1, sg2, ss0, ss1, ss2, si):
    xbufs = (xb0, xb1, xb2)
    rbufs = (rb0, rb1, rb2)
    sx = (sx0, sx1, sx2)
    sg = (sg0, sg1, sg2)
    ss = (ss0, ss1, ss2)

    wid = lax.axis_index("s") * _NC + lax.axis_index("c")
    b = wid // _WPR
    s = wid % _WPR

    seg_base = wid * _SEG
    p0 = s * _SEG

    def x_load(ch):
        sl = ch % _NBUF
        return pltpu.async_copy(
            x_hbm.at[pl.ds(seg_base + ch * _CHUNK, _CHUNK)], xbufs[sl], sx[sl])

    def g_load(ch):
        sl = ch % _NBUF
        return pltpu.async_copy(
            table_hbm.at[pid_v.at[pl.ds(p0 + ch * _CHUNK, _CHUNK)]],
            rbufs[sl], sg[sl])

    d_ids = pltpu.async_copy(ids_hbm.at[b], ids_v, si)
    dx01 = [x_load(0), x_load(1)]
    d_ids.wait()

    def cnt_step(j, carry):
        v = ids_v[pl.ds(j * _L, _L)]
        return carry + plsc.all_reduce_population_count(v != 0)

    cnt = lax.fori_loop(0, s * (_SEG // _L), cnt_step,
                        jnp.zeros((_L,), jnp.int32))

    def pid_step(j, carry):
        v = ids_v[pl.ds(j * _L, _L)]
        m = jnp.where(v != 0, jnp.int32(1), jnp.int32(0))
        c = plsc.cumsum(m) + carry
        pid_v[pl.ds(j * _L, _L)] = c * m
        return jnp.max(c)

    lax.fori_loop(s * (_SEG // _L), (s + 1) * (_SEG // _L), pid_step,
                  jnp.max(cnt))

    def accumulate(sl):
        xb, rb = xbufs[sl], rbufs[sl]

        def row_body(r, _):
            @plsc.parallel_loop(0, _D // _L, unroll=8)
            def _k(k):
                plsc.addupdate(xb.at[r, pl.ds(k * _L, _L)],
                               rb[r, pl.ds(k * _L, _L)])
            return 0

        lax.fori_loop(0, _CHUNK, row_body, 0)

    l_pend = [None] * _NBUF
    s_pend = [None] * _NBUF
    for ch in range(2):
        l_pend[ch % _NBUF] = (dx01[ch], g_load(ch))
    for ch in range(_NCHUNK):
        sl = ch % _NBUF
        dx, dg = l_pend[sl]
        dx.wait()
        dg.wait()
        l_pend[sl] = None
        if ch + 2 < _NCHUNK:
            s2 = (ch + 2) % _NBUF
            if s_pend[s2] is not None:
                s_pend[s2].wait()
                s_pend[s2] = None
            l_pend[s2] = (x_load(ch + 2), g_load(ch + 2))
        s_pend[sl] = pltpu.async_copy(
            xbufs[sl], out_hbm.at[pl.ds(seg_base + ch * _CHUNK, _CHUNK)],
            ss[sl])
    for d in s_pend:
        if d is not None:
            d.wait()


_pe = functools.partial(
    pl.kernel,
    out_type=jax.ShapeDtypeStruct((_ROWS, _D), jnp.float32),
    mesh=plsc.VectorSubcoreMesh(
        core_axis_name="c", subcore_axis_name="s",
        num_cores=_NC, num_subcores=_NS),
    compiler_params=pltpu.CompilerParams(needs_layout_passes=False),
    scratch_types=[
        pltpu.VMEM((_S,), jnp.int32),
        pltpu.VMEM((_S,), jnp.int32),
        pltpu.VMEM((_CHUNK, _D), jnp.float32),
        pltpu.VMEM((_CHUNK, _D), jnp.float32),
        pltpu.VMEM((_CHUNK, _D), jnp.float32),
        pltpu.VMEM((_CHUNK, _D), jnp.float32),
        pltpu.VMEM((_CHUNK, _D), jnp.float32),
        pltpu.VMEM((_CHUNK, _D), jnp.float32),
        pltpu.SemaphoreType.DMA,
        pltpu.SemaphoreType.DMA,
        pltpu.SemaphoreType.DMA,
        pltpu.SemaphoreType.DMA,
        pltpu.SemaphoreType.DMA,
        pltpu.SemaphoreType.DMA,
        pltpu.SemaphoreType.DMA,
        pltpu.SemaphoreType.DMA,
        pltpu.SemaphoreType.DMA,
        pltpu.SemaphoreType.DMA,
    ],
)(_body)


def kernel(X, input_ids, table):
    out = _pe(X.reshape(_ROWS, _D), input_ids, table)
    return out.reshape(_B, _S, _D)
